# baseline (speedup 1.0000x reference)
import functools

import jax
import jax.numpy as jnp
from jax import lax
from jax.experimental import pallas as pl
from jax.experimental.pallas import tpu as pltpu
from jax.experimental.pallas import tpu_sc as plsc


_N_TOK = 9216
_BLK = 1024
_GRID = _N_TOK // _BLK
_K = 1024
_D = 64
_LANES = 128
_IDX_ROWS = _N_TOK // _LANES
_ROWS_PER_BLK = _BLK // _LANES


def _argmin_body(flat_ref, cb_ref, w_ref, b_ref, idx_ref, cw_ref):
    flat = flat_ref[...]
    cb = cb_ref[...]
    s = lax.dot_general(flat, cb, (((1,), (1,)), ((), ())),
                        preferred_element_type=jnp.float32)
    rn = jnp.sum(flat * flat, axis=1, keepdims=True)
    cn = jnp.sum(cb * cb, axis=1)[None, :]
    d2 = rn - 2.0 * s + cn
    m = jnp.min(d2, axis=1, keepdims=True)
    iota_f = lax.broadcasted_iota(jnp.int32, (_BLK, _K), 1).astype(jnp.float32)
    idxf = jnp.min(jnp.where(d2 == m, iota_f, float(_K)), axis=1)
    idx_ref[...] = idxf.astype(jnp.int32).reshape(_ROWS_PER_BLK, _LANES)

    @pl.when(pl.program_id(0) == 0)
    def _():
        cw = lax.dot_general(cb, w_ref[...], (((1,), (1,)), ((), ())),
                             preferred_element_type=jnp.float32)
        cw_ref[...] = cw + b_ref[...]


def _tc_argmin(flat, cb, w2, b2):
    return pl.pallas_call(
        _argmin_body,
        grid=(_GRID,),
        in_specs=[
            pl.BlockSpec((_BLK, _D), lambda i: (i, 0)),
            pl.BlockSpec((_K, _D), lambda i: (0, 0)),
            pl.BlockSpec((2 * _D, _D), lambda i: (0, 0)),
            pl.BlockSpec((1, 2 * _D), lambda i: (0, 0)),
        ],
        out_specs=[
            pl.BlockSpec((_ROWS_PER_BLK, _LANES), lambda i: (i, 0)),
            pl.BlockSpec((_K, 2 * _D), lambda i: (0, 0)),
        ],
        out_shape=[
            jax.ShapeDtypeStruct((_IDX_ROWS, _LANES), jnp.int32),
            jax.ShapeDtypeStruct((_K, 2 * _D), jnp.float32),
        ],
    )(flat, cb, w2, b2)


_NC, _NS = 2, 16
_NW = _NC * _NS
_CHUNK = _LANES
_MAXCH = 3

_sc_mesh = plsc.VectorSubcoreMesh(core_axis_name="c", subcore_axis_name="s")


@functools.partial(
    pl.kernel,
    out_type=jax.ShapeDtypeStruct((_N_TOK, 2 * _D), jnp.float32),
    mesh=_sc_mesh,
    scratch_types=[
        pltpu.VMEM((8, _LANES), jnp.int32),
        pltpu.VMEM((_MAXCH, _CHUNK, 2 * _D), jnp.float32),
        pltpu.SemaphoreType.DMA,
    ],
    compiler_params=pltpu.CompilerParams(
        use_tc_tiling_on_sc=True,
        disable_bounds_checks=True,
        disable_semaphore_checks=True,
    ),
)
def _sc_gather(cw_hbm, idx_hbm, out_hbm, idx_v, rows_v, sem):
    wid = lax.axis_index("s") * _NC + lax.axis_index("c")
    rows = [wid, wid + 32, jnp.where(wid < 8, wid + 64, wid - 8)]
    copies = []
    for j, r in enumerate(rows):
        pltpu.sync_copy(idx_hbm.at[r], idx_v.at[j])
        copies.append(
            pltpu.async_copy(cw_hbm.at[idx_v.at[j]], rows_v.at[j], sem))
    for j, (r, c) in enumerate(zip(rows, copies)):
        c.wait()
        pltpu.sync_copy(rows_v.at[j], out_hbm.at[pl.ds(r * _CHUNK, _CHUNK)])


def kernel(z, codebook, W, b):
    B, T, D = z.shape
    flat = z.reshape(-1, D)
    w2 = jnp.concatenate([W, W], axis=0)
    b2 = jnp.concatenate([b, b]).reshape(1, 2 * D)
    idx, cw = _tc_argmin(flat, codebook, w2, b2)
    q = _sc_gather(cw, idx)
    return q[:, :D].reshape(B, T, D)


# speedup vs baseline: 1.1122x; 1.1122x over previous
import functools

import jax
import jax.numpy as jnp
from jax import lax
from jax.experimental import pallas as pl
from jax.experimental.pallas import tpu as pltpu
from jax.experimental.pallas import tpu_sc as plsc


_N_TOK = 9216
_B, _T = 16, 576
_BB = 2
_BLK = _BB * _T
_GRID = _B // _BB
_K = 1024
_D = 64


def _argmin_body(zt_ref, cb_ref, w_ref, b_ref, idx_ref, cw_ref):
    cb = cb_ref[...]
    cn = jnp.sum(cb * cb, axis=1)[None, :]
    iota_f = lax.broadcasted_iota(jnp.int32, (_T, _K), 1).astype(jnp.float32)
    for i in range(_BB):
        flat = zt_ref[i].T
        s = lax.dot_general(flat, cb, (((1,), (1,)), ((), ())),
                            preferred_element_type=jnp.float32)
        rn = jnp.sum(flat * flat, axis=1, keepdims=True)
        d2 = rn - 2.0 * s + cn
        m = jnp.min(d2, axis=1, keepdims=True)
        idxf = jnp.min(jnp.where(d2 == m, iota_f, float(_K)), axis=1)
        idx_ref[0, i, :] = idxf.astype(jnp.int32)

    @pl.when(pl.program_id(0) == 0)
    def _():
        cw = lax.dot_general(cb, w_ref[...], (((1,), (1,)), ((), ())),
                             preferred_element_type=jnp.float32)
        cw_ref[...] = cw + b_ref[...]


def _tc_argmin(zt, cb, w, b2d):
    return pl.pallas_call(
        _argmin_body,
        grid=(_GRID,),
        in_specs=[
            pl.BlockSpec((_BB, _D, _T), lambda i: (i, 0, 0)),
            pl.BlockSpec((_K, _D), lambda i: (0, 0)),
            pl.BlockSpec((_D, _D), lambda i: (0, 0)),
            pl.BlockSpec((1, _D), lambda i: (0, 0)),
        ],
        out_specs=[
            pl.BlockSpec((1, _BB, _T), lambda i: (i, 0, 0)),
            pl.BlockSpec((_K, _D), lambda i: (0, 0)),
        ],
        out_shape=[
            jax.ShapeDtypeStruct((_GRID, _BB, _T), jnp.int32),
            jax.ShapeDtypeStruct((_K, _D), jnp.float32),
        ],
    )(zt, cb, w, b2d)


_NC, _NS = 2, 16
_NW = _NC * _NS
_CHUNK = 96
_NCHUNK = _N_TOK // (_NW * _CHUNK)
_BPW = _NCHUNK * _CHUNK

_sc_mesh = plsc.VectorSubcoreMesh(core_axis_name="c", subcore_axis_name="s")


@functools.partial(
    pl.kernel,
    out_type=jax.ShapeDtypeStruct((_N_TOK, _D), jnp.float32),
    mesh=_sc_mesh,
    scratch_types=[
        pltpu.VMEM((_NCHUNK, _CHUNK), jnp.int32),
        pltpu.VMEM((_NCHUNK, _CHUNK, _D), jnp.float32),
        pltpu.SemaphoreType.DMA,
    ],
    compiler_params=pltpu.CompilerParams(
        use_tc_tiling_on_sc=False,
        disable_bounds_checks=True,
        disable_semaphore_checks=True,
    ),
)
def _sc_gather(cw_hbm, idx_hbm, out_hbm, idx_v, rows_v, sem):
    wid = lax.axis_index("s") * _NC + lax.axis_index("c")
    base = wid * _BPW
    pltpu.sync_copy(idx_hbm.at[wid], idx_v)
    copies = []
    for j in range(_NCHUNK):
        copies.append(
            pltpu.async_copy(cw_hbm.at[idx_v.at[j]], rows_v.at[j], sem))
    for j, c in enumerate(copies):
        c.wait()
        pltpu.sync_copy(rows_v.at[j],
                        out_hbm.at[pl.ds(base + j * _CHUNK, _CHUNK)])


def kernel(z, codebook, W, b):
    B, T, D = z.shape
    zt = jnp.swapaxes(z, 1, 2)
    idx, cw = _tc_argmin(zt, codebook, W, b.reshape(1, D))
    idx = idx.reshape(_NW, _NCHUNK, _CHUNK)
    q = _sc_gather(cw, idx)
    return q.reshape(B, T, D)


# speedup vs baseline: 1.1434x; 1.0281x over previous
import functools

import jax
import jax.numpy as jnp
from jax import lax
from jax.experimental import pallas as pl
from jax.experimental.pallas import tpu as pltpu
from jax.experimental.pallas import tpu_sc as plsc


_N_TOK = 9216
_B, _T = 16, 576
_BB = 4
_BLK = _BB * _T
_GRID = _B // _BB
_K = 1024
_D = 64


def _argmin_body(zt_ref, cb_ref, w_ref, b_ref, idx_ref, cw_ref):
    cb = cb_ref[...]
    cn = jnp.sum(cb * cb, axis=1)[None, :]
    iota_f = lax.broadcasted_iota(jnp.int32, (_T, _K), 1).astype(jnp.float32)
    for i in range(_BB):
        flat = zt_ref[i].T
        s = lax.dot_general(flat, cb, (((1,), (1,)), ((), ())),
                            preferred_element_type=jnp.float32)
        rn = jnp.sum(flat * flat, axis=1, keepdims=True)
        d2 = rn - 2.0 * s + cn
        m = jnp.min(d2, axis=1, keepdims=True)
        idxf = jnp.min(jnp.where(d2 == m, iota_f, float(_K)), axis=1)
        idx_ref[0, i, :] = idxf.astype(jnp.int32)

    @pl.when(pl.program_id(0) == 0)
    def _():
        cw = lax.dot_general(cb, w_ref[...], (((1,), (1,)), ((), ())),
                             preferred_element_type=jnp.float32)
        cw_ref[...] = cw + b_ref[...]


def _tc_argmin(zt, cb, w, b2d):
    return pl.pallas_call(
        _argmin_body,
        grid=(_GRID,),
        in_specs=[
            pl.BlockSpec((_BB, _D, _T), lambda i: (i, 0, 0)),
            pl.BlockSpec((_K, _D), lambda i: (0, 0)),
            pl.BlockSpec((_D, _D), lambda i: (0, 0)),
            pl.BlockSpec((1, _D), lambda i: (0, 0)),
        ],
        out_specs=[
            pl.BlockSpec((1, _BB, _T), lambda i: (i, 0, 0)),
            pl.BlockSpec((_K, _D), lambda i: (0, 0)),
        ],
        out_shape=[
            jax.ShapeDtypeStruct((_GRID, _BB, _T), jnp.int32),
            jax.ShapeDtypeStruct((_K, _D), jnp.float32),
        ],
    )(zt, cb, w, b2d)


_NC, _NS = 2, 16
_NW = _NC * _NS
_CHUNK = 96
_NCHUNK = _N_TOK // (_NW * _CHUNK)
_BPW = _NCHUNK * _CHUNK

_sc_mesh = plsc.VectorSubcoreMesh(core_axis_name="c", subcore_axis_name="s")


@functools.partial(
    pl.kernel,
    out_type=jax.ShapeDtypeStruct((_N_TOK, _D), jnp.float32),
    mesh=_sc_mesh,
    scratch_types=[
        pltpu.VMEM((_NCHUNK, _CHUNK), jnp.int32),
        pltpu.VMEM((_NCHUNK, _CHUNK, _D), jnp.float32),
        pltpu.SemaphoreType.DMA,
    ],
    compiler_params=pltpu.CompilerParams(
        use_tc_tiling_on_sc=False,
        disable_bounds_checks=True,
        disable_semaphore_checks=True,
    ),
)
def _sc_gather(cw_hbm, idx_hbm, out_hbm, idx_v, rows_v, sem):
    wid = lax.axis_index("s") * _NC + lax.axis_index("c")
    base = wid * _BPW
    pltpu.sync_copy(idx_hbm.at[wid], idx_v)
    copies = []
    for j in range(_NCHUNK):
        copies.append(
            pltpu.async_copy(cw_hbm.at[idx_v.at[j]], rows_v.at[j], sem))
    for j, c in enumerate(copies):
        c.wait()
        pltpu.sync_copy(rows_v.at[j],
                        out_hbm.at[pl.ds(base + j * _CHUNK, _CHUNK)])


def kernel(z, codebook, W, b):
    B, T, D = z.shape
    zt = jnp.swapaxes(z, 1, 2)
    idx, cw = _tc_argmin(zt, codebook, W, b.reshape(1, D))
    idx = idx.reshape(_NW, _NCHUNK, _CHUNK)
    q = _sc_gather(cw, idx)
    return q.reshape(B, T, D)


# speedup vs baseline: 1.1547x; 1.0099x over previous
import functools

import jax
import jax.numpy as jnp
from jax import lax
from jax.experimental import pallas as pl
from jax.experimental.pallas import tpu as pltpu
from jax.experimental.pallas import tpu_sc as plsc


_N_TOK = 9216
_B, _T = 16, 576
_BB = 8
_BLK = _BB * _T
_GRID = _B // _BB
_K = 1024
_D = 64


def _argmin_body(zt_ref, cb_ref, w_ref, b_ref, idx_ref, cw_ref):
    cb = cb_ref[...]
    cn = jnp.sum(cb * cb, axis=1)[None, :]
    iota_f = lax.broadcasted_iota(jnp.int32, (_T, _K), 1).astype(jnp.float32)
    for i in range(_BB):
        flat = zt_ref[i].T
        s = lax.dot_general(flat, cb, (((1,), (1,)), ((), ())),
                            preferred_element_type=jnp.float32)
        rn = jnp.sum(flat * flat, axis=1, keepdims=True)
        d2 = rn - 2.0 * s + cn
        m = jnp.min(d2, axis=1, keepdims=True)
        idxf = jnp.min(jnp.where(d2 == m, iota_f, float(_K)), axis=1)
        idx_ref[0, i, :] = idxf.astype(jnp.int32)

    @pl.when(pl.program_id(0) == 0)
    def _():
        cw = lax.dot_general(cb, w_ref[...], (((1,), (1,)), ((), ())),
                             preferred_element_type=jnp.float32)
        cw_ref[...] = cw + b_ref[...]


def _tc_argmin(zt, cb, w, b2d):
    return pl.pallas_call(
        _argmin_body,
        grid=(_GRID,),
        in_specs=[
            pl.BlockSpec((_BB, _D, _T), lambda i: (i, 0, 0)),
            pl.BlockSpec((_K, _D), lambda i: (0, 0)),
            pl.BlockSpec((_D, _D), lambda i: (0, 0)),
            pl.BlockSpec((1, _D), lambda i: (0, 0)),
        ],
        out_specs=[
            pl.BlockSpec((1, _BB, _T), lambda i: (i, 0, 0)),
            pl.BlockSpec((_K, _D), lambda i: (0, 0)),
        ],
        out_shape=[
            jax.ShapeDtypeStruct((_GRID, _BB, _T), jnp.int32),
            jax.ShapeDtypeStruct((_K, _D), jnp.float32),
        ],
    )(zt, cb, w, b2d)


_NC, _NS = 2, 16
_NW = _NC * _NS
_CHUNK = 96
_NCHUNK = _N_TOK // (_NW * _CHUNK)
_BPW = _NCHUNK * _CHUNK

_sc_mesh = plsc.VectorSubcoreMesh(core_axis_name="c", subcore_axis_name="s")


@functools.partial(
    pl.kernel,
    out_type=jax.ShapeDtypeStruct((_N_TOK, _D), jnp.float32),
    mesh=_sc_mesh,
    scratch_types=[
        pltpu.VMEM((_NCHUNK, _CHUNK), jnp.int32),
        pltpu.VMEM((_NCHUNK, _CHUNK, _D), jnp.float32),
        pltpu.SemaphoreType.DMA,
    ],
    compiler_params=pltpu.CompilerParams(
        use_tc_tiling_on_sc=False,
        disable_bounds_checks=True,
        disable_semaphore_checks=True,
    ),
)
def _sc_gather(cw_hbm, idx_hbm, out_hbm, idx_v, rows_v, sem):
    wid = lax.axis_index("s") * _NC + lax.axis_index("c")
    base = wid * _BPW
    pltpu.sync_copy(idx_hbm.at[wid], idx_v)
    copies = []
    for j in range(_NCHUNK):
        copies.append(
            pltpu.async_copy(cw_hbm.at[idx_v.at[j]], rows_v.at[j], sem))
    for j, c in enumerate(copies):
        c.wait()
        pltpu.sync_copy(rows_v.at[j],
                        out_hbm.at[pl.ds(base + j * _CHUNK, _CHUNK)])


def kernel(z, codebook, W, b):
    B, T, D = z.shape
    zt = jnp.swapaxes(z, 1, 2)
    idx, cw = _tc_argmin(zt, codebook, W, b.reshape(1, D))
    idx = idx.reshape(_NW, _NCHUNK, _CHUNK)
    q = _sc_gather(cw, idx)
    return q.reshape(B, T, D)


# speedup vs baseline: 1.1548x; 1.0001x over previous
import functools

import jax
import jax.numpy as jnp
from jax import lax
from jax.experimental import pallas as pl
from jax.experimental.pallas import tpu as pltpu
from jax.experimental.pallas import tpu_sc as plsc


_N_TOK = 9216
_B, _T = 16, 576
_BB = 16
_BLK = _BB * _T
_GRID = _B // _BB
_K = 1024
_D = 64


def _argmin_body(zt_ref, cb_ref, w_ref, b_ref, idx_ref, cw_ref):
    cb = cb_ref[...]
    cn = jnp.sum(cb * cb, axis=1)[None, :]
    iota_f = lax.broadcasted_iota(jnp.int32, (_T, _K), 1).astype(jnp.float32)
    for i in range(_BB):
        flat = zt_ref[i].T
        s = lax.dot_general(flat, cb, (((1,), (1,)), ((), ())),
                            preferred_element_type=jnp.float32)
        rn = jnp.sum(flat * flat, axis=1, keepdims=True)
        d2 = rn - 2.0 * s + cn
        m = jnp.min(d2, axis=1, keepdims=True)
        idxf = jnp.min(jnp.where(d2 == m, iota_f, float(_K)), axis=1)
        idx_ref[0, i, :] = idxf.astype(jnp.int32)

    @pl.when(pl.program_id(0) == 0)
    def _():
        cw = lax.dot_general(cb, w_ref[...], (((1,), (1,)), ((), ())),
                             preferred_element_type=jnp.float32)
        cw_ref[...] = cw + b_ref[...]


def _tc_argmin(zt, cb, w, b2d):
    return pl.pallas_call(
        _argmin_body,
        grid=(_GRID,),
        in_specs=[
            pl.BlockSpec((_BB, _D, _T), lambda i: (i, 0, 0)),
            pl.BlockSpec((_K, _D), lambda i: (0, 0)),
            pl.BlockSpec((_D, _D), lambda i: (0, 0)),
            pl.BlockSpec((1, _D), lambda i: (0, 0)),
        ],
        out_specs=[
            pl.BlockSpec((1, _BB, _T), lambda i: (i, 0, 0)),
            pl.BlockSpec((_K, _D), lambda i: (0, 0)),
        ],
        out_shape=[
            jax.ShapeDtypeStruct((_GRID, _BB, _T), jnp.int32),
            jax.ShapeDtypeStruct((_K, _D), jnp.float32),
        ],
    )(zt, cb, w, b2d)


_NC, _NS = 2, 16
_NW = _NC * _NS
_CHUNK = 96
_NCHUNK = _N_TOK // (_NW * _CHUNK)
_BPW = _NCHUNK * _CHUNK

_sc_mesh = plsc.VectorSubcoreMesh(core_axis_name="c", subcore_axis_name="s")


@functools.partial(
    pl.kernel,
    out_type=jax.ShapeDtypeStruct((_N_TOK, _D), jnp.float32),
    mesh=_sc_mesh,
    scratch_types=[
        pltpu.VMEM((_NCHUNK, _CHUNK), jnp.int32),
        pltpu.VMEM((_NCHUNK, _CHUNK, _D), jnp.float32),
        pltpu.SemaphoreType.DMA,
    ],
    compiler_params=pltpu.CompilerParams(
        use_tc_tiling_on_sc=False,
        disable_bounds_checks=True,
        disable_semaphore_checks=True,
    ),
)
def _sc_gather(cw_hbm, idx_hbm, out_hbm, idx_v, rows_v, sem):
    wid = lax.axis_index("s") * _NC + lax.axis_index("c")
    base = wid * _BPW
    pltpu.sync_copy(idx_hbm.at[wid], idx_v)
    copies = []
    for j in range(_NCHUNK):
        copies.append(
            pltpu.async_copy(cw_hbm.at[idx_v.at[j]], rows_v.at[j], sem))
    for j, c in enumerate(copies):
        c.wait()
        pltpu.sync_copy(rows_v.at[j],
                        out_hbm.at[pl.ds(base + j * _CHUNK, _CHUNK)])


def kernel(z, codebook, W, b):
    B, T, D = z.shape
    zt = jnp.swapaxes(z, 1, 2)
    idx, cw = _tc_argmin(zt, codebook, W, b.reshape(1, D))
    idx = idx.reshape(_NW, _NCHUNK, _CHUNK)
    q = _sc_gather(cw, idx)
    return q.reshape(B, T, D)
